# baseline (speedup 1.0000x reference)
import functools

import jax
import jax.numpy as jnp
from jax import lax
from jax.experimental import pallas as pl
from jax.experimental.pallas import tpu as pltpu
from jax.experimental.pallas import tpu_sc as plsc

NC = 2
NS = 16
L = 16
NW = NC * NS

B = 16384
D = 64
DP = 128
V = 1000000
CB = 128
CPT = 245
TAIL = (V // CB) * CB
BPW = B // NW
DUMMY = B

_params = pltpu.CompilerParams(
    needs_layout_passes=False, use_tc_tiling_on_sc=True)

_mesh = plsc.VectorSubcoreMesh(
    core_axis_name="c", subcore_axis_name="s",
    num_cores=NC, num_subcores=NS)


def _sweep_body(tT, tail_hbm, idx_hbm, out,
                ixall, lcomb, pcomb,
                buf0, buf1, tbuf, stage, spos, sem0, sem1, semt, sems):
    wid = lax.axis_index("s") * NC + lax.axis_index("c")
    lane = lax.iota(jnp.int32, L)

    lo = wid * (CPT * CB)
    hi = jnp.minimum(lo + CPT * CB, V)
    nch = (hi - lo) // CB

    ctail = pltpu.async_copy(tail_hbm, tbuf, semt)
    pltpu.sync_copy(idx_hbm, ixall)

    def scanstep(g, off):
        v = ixall[pl.ds(g * L, L)]
        b = g * L + lane
        m = (v >= lo) & (v < hi)
        packed = ((v - lo) << 14) | b
        plsc.store_compressed(lcomb.at[pl.ds(off, L)], packed, mask=m)
        return off + plsc.all_reduce_population_count(m)[0]

    n = lax.fori_loop(0, B // L, scanstep, jnp.int32(0))
    ngrp = (n + (L - 1)) // L

    def compress_chunk(a):
        a_rel = a - lo

        def rescan(g, p):
            k16 = pl.ds(g * L, L)
            packed = lcomb[k16]
            vr = packed >> 14
            m = ((g * L + lane) < n) & (vr >= a_rel) & (vr < a_rel + CB)
            plsc.store_compressed(pcomb.at[pl.ds(p, L)], packed, mask=m)
            return p + plsc.all_reduce_population_count(m)[0]

        return lax.fori_loop(0, ngrp, rescan, jnp.int32(0))

    def extract_scatter(bufref, a, pc):
        a_rel = a - lo

        def egroup(e, _):
            packed = pcomb[pl.ds(e * L, L)]
            bv = packed & (2 ** 14 - 1)
            live = (e * L + lane) < pc
            spos[...] = jnp.where(live, bv, DUMMY)
            cols = (packed >> 14) - a_rel
            for k in range(L):
                @pl.when((e * L + k) < pc)
                def _():
                    col = jnp.full((L,), cols[k], jnp.int32)
                    for m4 in range(D // L):
                        rows = m4 * L + lane
                        g16 = plsc.load_gather(bufref, [rows, col])
                        stage[k, pl.ds(m4 * L, L)] = g16
            pltpu.async_copy(stage, out.at[spos], sems).wait()
            return 0

        lax.fori_loop(0, (pc + (L - 1)) // L, egroup, 0)

    pltpu.async_copy(tT.at[:, pl.ds(lo, CB)], buf0, sem0)

    @pl.when(nch > 1)
    def _():
        pltpu.async_copy(tT.at[:, pl.ds(lo + CB, CB)], buf1, sem1)

    def pairstep(p, _):
        for q, (bufq, semq) in ((0, (buf0, sem0)), (1, (buf1, sem1))):
            ci = 2 * p + q

            @pl.when(ci < nch)
            def _(bufq=bufq, semq=semq, ci=ci):
                a = lo + ci * CB
                pltpu.make_async_copy(tT.at[:, pl.ds(0, CB)], bufq, semq).wait()
                pc = compress_chunk(a)
                extract_scatter(bufq, a, pc)

                @pl.when(ci + 2 < nch)
                def _():
                    pltpu.async_copy(
                        tT.at[:, pl.ds(a + 2 * CB, CB)], bufq, semq)
        return 0

    lax.fori_loop(0, (nch + 1) // 2, pairstep, 0)

    ctail.wait()
    pc_t = compress_chunk(jnp.int32(TAIL))
    extract_scatter(tbuf, jnp.int32(TAIL), pc_t)


def _dots_body(ue_hbm, ie_hbm, out_hbm, ub, ib, dots, sem):
    wid = lax.axis_index("s") * NC + lax.axis_index("c")
    base = wid * BPW
    lane = lax.iota(jnp.int32, L)

    HB = BPW // 2
    for h in range(2):
        cu = pltpu.async_copy(ue_hbm.at[pl.ds(base + h * HB, HB)], ub, sem)
        ci = pltpu.async_copy(ie_hbm.at[pl.ds(base + h * HB, HB)], ib, sem)
        cu.wait()
        ci.wait()

        def group(g, _):
            rows = g * L + lane
            acc = jnp.zeros((L,), jnp.float32)
            for j in range(D):
                col = jnp.full((L,), j, jnp.int32)
                u = plsc.load_gather(ub, [rows, col])
                v = plsc.load_gather(ib, [rows, col])
                acc = acc + u * v
            dots[pl.ds(h * HB + g * L, L)] = acc
            return 0

        lax.fori_loop(0, HB // L, group, 0)
    pltpu.sync_copy(dots, out_hbm.at[pl.ds(base, BPW)])


_sweep = functools.partial(
    pl.kernel,
    out_type=jax.ShapeDtypeStruct((B + L, DP), jnp.float32),
    mesh=_mesh,
    compiler_params=_params,
    scratch_types=[
        pltpu.VMEM((B,), jnp.int32),
        pltpu.VMEM((B + L,), jnp.int32),
        pltpu.VMEM((B + L,), jnp.int32),
        pltpu.VMEM((D, CB), jnp.float32),
        pltpu.VMEM((D, CB), jnp.float32),
        pltpu.VMEM((D, CB), jnp.float32),
        pltpu.VMEM((L, DP), jnp.float32),
        pltpu.VMEM((L,), jnp.int32),
        pltpu.SemaphoreType.DMA,
        pltpu.SemaphoreType.DMA,
        pltpu.SemaphoreType.DMA,
        pltpu.SemaphoreType.DMA,
    ],
)(_sweep_body)

_dots = functools.partial(
    pl.kernel,
    out_type=jax.ShapeDtypeStruct((B,), jnp.float32),
    mesh=_mesh,
    compiler_params=_params,
    scratch_types=[
        pltpu.VMEM((BPW // 2, DP), jnp.float32),
        pltpu.VMEM((BPW // 2, DP), jnp.float32),
        pltpu.VMEM((BPW,), jnp.float32),
        pltpu.SemaphoreType.DMA,
    ],
)(_dots_body)


def kernel(x, user_table, item_table):
    uidx = x[:, 0]
    iidx = x[:, 1]
    zpad = jnp.zeros((D, CB - (V - TAIL)), jnp.float32)
    utail = jnp.concatenate([user_table[TAIL:].T, zpad], axis=1)
    itail = jnp.concatenate([item_table[TAIL:].T, zpad], axis=1)
    ue_pad = _sweep(user_table.T, utail, uidx)
    ie_pad = _sweep(item_table.T, itail, iidx)
    dots = _dots(ue_pad, ie_pad)
    return (dots[:, None], ue_pad[:B, :D], ie_pad[:B, :D])
